# baseline (speedup 1.0000x reference)
import jax
import jax.numpy as jnp
from jax import lax
from jax.experimental import pallas as pl
from jax.experimental.pallas import tpu as pltpu

CLS = 1000
N, K = 4096, 26
B = 64
NSLOT = 4
Q = 2
BQ = B // Q
STEPS = N // B


def _copy(buf, o_ref, sem, i, slot, q):
    return pltpu.make_async_copy(
        buf.at[slot, pl.ds(q * BQ, BQ)],
        o_ref.at[pl.ds(i * B + q * BQ, BQ)],
        sem.at[slot, q],
    )


def _onehot(x_ref, o_ref, buf, sem):
    iota = lax.broadcasted_iota(jnp.int32, (B, K, CLS), 2)

    def step(i, _):
        slot = lax.rem(i, NSLOT)

        @pl.when(i >= NSLOT)
        def _():
            for q in range(Q):
                _copy(buf, o_ref, sem, i - NSLOT, slot, q).wait()

        xb = x_ref[pl.ds(i * B, B), :]
        buf[slot] = (xb[:, :, None] == iota).astype(jnp.int32)
        for q in range(Q):
            _copy(buf, o_ref, sem, i, slot, q).start()
        return ()

    lax.fori_loop(0, STEPS, step, (), unroll=False)

    for j in range(NSLOT):
        i = STEPS - NSLOT + j
        for q in range(Q):
            _copy(buf, o_ref, sem, i, i % NSLOT, q).wait()


def kernel(x):
    return pl.pallas_call(
        _onehot,
        in_specs=[pl.BlockSpec(memory_space=pltpu.VMEM)],
        out_specs=pl.BlockSpec(memory_space=pltpu.MemorySpace.HBM),
        out_shape=jax.ShapeDtypeStruct((N, K, CLS), jnp.int32),
        scratch_shapes=[
            pltpu.VMEM((NSLOT, B, K, CLS), jnp.int32),
            pltpu.SemaphoreType.DMA((NSLOT, Q)),
        ],
    )(x)
